# baseline (speedup 1.0000x reference)
import functools

import jax
import jax.numpy as jnp
import numpy as np
from jax.experimental import pallas as pl
from jax.experimental.pallas import tpu as pltpu
from jax.experimental.pallas import tpu_sc as plsc

N = 10000
E = 160000
B = 50
C = 32
L = 9
H = 8
DA = 16
DV = 2
NL = 4
DE = 64
HT = 64

f32 = jnp.float32
i32 = jnp.int32

NCORES = 2
NSUB = 16
NTILES = NCORES * NSUB
CHUNK = 128
NCH = E // CHUNK
SUPER = 2
SEDGE = SUPER * CHUNK
NSS = E // SEDGE
KAGG = (NSS + NSUB - 1) // NSUB
NGS = 2 * NSS
KGS = (NGS + NTILES - 1) // NTILES
STRIPE = N // NSUB
HROW = 80

NB = 2000
EB = 8000


def _dot(a, b):
    return jnp.dot(a, b, preferred_element_type=f32)


def _embed_body(t_ref, kap_ref, W1_ref, b1_ref, W2_ref, b2_ref, We_ref,
                be_ref, seg_ref, res_ref, WA1_ref, WA2_ref, WB1_ref,
                WB2_ref, ones_ref, l0_ref, v2_ref):
    tp = (2.0 * np.pi) * t_ref[:, :] * kap_ref[:, :]
    four = jnp.concatenate([jnp.cos(tp), jnp.sin(tp)], axis=1)
    h1 = jnp.maximum(_dot(four, W1_ref[:, :]) + b1_ref[:, :], 0.0)
    et = jnp.maximum(_dot(h1, W2_ref[:, :]) + b2_ref[:, :], 0.0)
    seg = seg_ref[:, :]
    oh = (seg == jax.lax.broadcasted_iota(i32, (seg.shape[0], 56), 1)
          ).astype(f32)
    tn = _dot(oh, et)
    l0 = _dot(tn, We_ref[:, :]) + be_ref[:, :]
    l0_ref[:, :] = l0
    res = res_ref[:, :]
    v2_ref[0, :, :] = _dot(res, WA1_ref[:, :]) + _dot(l0, WB1_ref[:, :])
    v2_ref[1, :, :] = (_dot(res, WA2_ref[:, :]) + _dot(l0, WB2_ref[:, :])
                       + ones_ref[:, :])


def _e1_body(ssrc_ref, sdst_ref, ef_ref, Wq_ref, Wk_ref, Wea_ref, S_ref,
             out_ref):
    q = _dot(sdst_ref[:, :].astype(f32), Wq_ref[:, :])
    k = _dot(ssrc_ref[:, :].astype(f32), Wk_ref[:, :])
    lg = _dot(q * k, S_ref[:, :]) + _dot(ef_ref[:, :], Wea_ref[:, :])
    out_ref[:, :] = jnp.exp(lg)


def _ef_body(ef_ref, ssrc_ref, sdst_ref, WeA_ref, WeB_ref, WeC_ref,
             Wq_ref, Wk_ref, Wea_ref, S_ref, efo_ref, exx_ref):
    ssrc = ssrc_ref[:, :].astype(f32)
    sdst = sdst_ref[:, :].astype(f32)
    efn = jnp.maximum(_dot(ef_ref[:, :], WeA_ref[:, :])
                      + _dot(ssrc, WeB_ref[:, :])
                      + _dot(sdst, WeC_ref[:, :]), 0.0)
    efo_ref[:, :] = efn
    q = _dot(sdst, Wq_ref[:, :])
    k = _dot(ssrc, Wk_ref[:, :])
    lg = _dot(q * k, S_ref[:, :]) + _dot(efn, Wea_ref[:, :])
    exx_ref[:, :] = jnp.exp(lg)


def _n2_body(numA_ref, numB_ref, node_ref, res_ref,
             SELA_ref, SELB_ref, WoA_ref, WoB_ref, G32_ref, Hx_ref,
             gam_ref, W1f_ref, T64_ref, W2f_ref, WrA_ref, WrB_ref,
             VA1_ref, VA2_ref, VB1_ref, VB2_ref, ones_ref,
             rout_ref, nout_ref, sa_ref, sb_ref, v2_ref):
    numA = numA_ref[:, :]
    numB = numB_ref[:, :]
    dentA = _dot(numB, SELA_ref[:, :]) + 1e-9
    dentB = _dot(numB, SELB_ref[:, :]) + 1e-9
    x = (node_ref[:, :] + _dot(numA / dentA, WoA_ref[:, :])
         + _dot(numB / dentB, WoB_ref[:, :]))
    ms = _dot(x * x, G32_ref[:, :])
    rinv = jax.lax.rsqrt(ms + 1e-6)
    xn = x * _dot(rinv, Hx_ref[:, :]) * gam_ref[:, :]
    h = _dot(xn, W1f_ref[:, :])
    g = jax.nn.sigmoid(h[:, :2 * C])
    hg = h * _dot(g, T64_ref[:, :])
    nn = xn + _dot(hg, W2f_ref[:, :])
    res = res_ref[:, :]
    rnew = res + _dot(res, WrA_ref[:, :]) + _dot(nn, WrB_ref[:, :])
    rout_ref[:, :] = rnew
    nout_ref[:, :] = nn
    sa_ref[:, :] = rnew[:, :C]
    sb_ref[:, :] = nn[:, :C]
    v2_ref[0, :, :] = _dot(rnew, VA1_ref[:, :]) + _dot(nn, VB1_ref[:, :])
    v2_ref[1, :, :] = (_dot(rnew, VA2_ref[:, :]) + _dot(nn, VB2_ref[:, :])
                       + ones_ref[:, :])


def _gs_body(s_hbm, sd_hbm, out_hbm, ib, db, sem):
    c = jax.lax.axis_index("c")
    sub = jax.lax.axis_index("s")
    w = sub * NCORES + c

    def fire(k, b):
        tid = k * NTILES + w

        @pl.when(tid < NGS)
        def _():
            pltpu.sync_copy(sd_hbm.at[pl.ds(tid * SUPER, SUPER)], ib.at[b])
            for j in range(SUPER):
                pltpu.async_copy(
                    s_hbm.at[ib.at[b].at[j]],
                    db.at[b].at[pl.ds(j * CHUNK, CHUNK)], sem.at[b])

    def drain(k, b):
        tid = k * NTILES + w

        @pl.when(tid < NGS)
        def _():
            for j in range(SUPER):
                pltpu.make_async_copy(
                    s_hbm.at[ib.at[b].at[j]],
                    db.at[b].at[pl.ds(j * CHUNK, CHUNK)], sem.at[b]).wait()
            pltpu.sync_copy(db.at[b], out_hbm.at[pl.ds(tid * SEDGE, SEDGE)])

    def pipe(m, carry):
        b = jax.lax.rem(m, 2)
        fire(m, b)

        @pl.when(m >= 1)
        def _():
            drain(m - 1, 1 - b)
        return carry

    jax.lax.fori_loop(0, KGS + 1, pipe, 0)


def _agg_body(v_hbm, exx_hbm, src_hbm, dst_hbm, zero_hbm, out_hbm,
              isb, idb, rb, eb, acc, sem):
    c = jax.lax.axis_index("c")
    sub = jax.lax.axis_index("s")

    pltpu.sync_copy(zero_hbm.at[pl.ds(sub * STRIPE, STRIPE)],
                    acc.at[pl.ds(sub * STRIPE, STRIPE)])
    plsc.subcore_barrier()
    base = c * N

    def fire(k, b):
        sid = k * NSUB + sub

        @pl.when(sid < NSS)
        def _():
            cid0 = sid * SUPER
            pltpu.sync_copy(src_hbm.at[pl.ds(cid0, SUPER)], isb.at[b])
            pltpu.sync_copy(dst_hbm.at[pl.ds(cid0, SUPER)], idb.at[b])
            for j in range(SUPER):
                for q in range(CHUNK // 16):
                    sl = pl.ds(q * 16, 16)
                    isb[b, j, sl] = isb[b, j, sl] + base
                pltpu.async_copy(
                    v_hbm.at[isb.at[b].at[j]],
                    rb.at[b].at[pl.ds(j * CHUNK, CHUNK)], sem.at[b])

    def process(k, b):
        sid = k * NSUB + sub

        @pl.when(sid < NSS)
        def _():
            pltpu.sync_copy(exx_hbm.at[pl.ds(sid * SEDGE, SEDGE)], eb)
            for j in range(SUPER):
                pltpu.make_async_copy(
                    v_hbm.at[isb.at[b].at[j]],
                    rb.at[b].at[pl.ds(j * CHUNK, CHUNK)], sem.at[b]).wait()

            def estep(e, cy):
                exr = eb[e, :]
                for kk in range(HROW // 16):
                    sl = pl.ds(kk * 16, 16)
                    rb[b, e, sl] = rb[b, e, sl] * exr
                return cy

            jax.lax.fori_loop(0, SEDGE, estep, 0)
            for j in range(SUPER):
                pltpu.sync_copy(rb.at[b].at[pl.ds(j * CHUNK, CHUNK)],
                                acc.at[idb.at[b].at[j]], add=True)

    def pipe(m, carry):
        b = jax.lax.rem(m, 2)
        fire(m, b)

        @pl.when(m >= 1)
        def _():
            process(m - 1, 1 - b)
        return carry

    jax.lax.fori_loop(0, KAGG + 1, pipe, 0)

    plsc.subcore_barrier()
    pltpu.sync_copy(acc.at[pl.ds(sub * STRIPE, STRIPE)],
                    out_hbm.at[pl.ds(c * N + sub * STRIPE, STRIPE)])


def _mesh():
    return plsc.VectorSubcoreMesh(core_axis_name="c", subcore_axis_name="s")


_SC_PARAMS = pltpu.CompilerParams(use_tc_tiling_on_sc=False)


def _gs_call(s, srcdst):
    f = pl.kernel(
        _gs_body,
        out_type=jax.ShapeDtypeStruct((2 * E, 2 * C), jnp.bfloat16),
        mesh=_mesh(),
        scratch_types=[
            pltpu.VMEM((2, SUPER, CHUNK), i32),
            pltpu.VMEM((2, SEDGE, 2 * C), jnp.bfloat16),
            pltpu.SemaphoreType.DMA((2,)),
        ],
        compiler_params=_SC_PARAMS,
    )
    return f(s, srcdst)


def _agg_call(vab, exx, src, dst, zeros80):
    f = pl.kernel(
        _agg_body,
        out_type=jax.ShapeDtypeStruct((2 * N, HROW), f32),
        mesh=_mesh(),
        scratch_types=[
            pltpu.VMEM((2, SUPER, CHUNK), i32),
            pltpu.VMEM((2, SUPER, CHUNK), i32),
            pltpu.VMEM((2, SEDGE, HROW), f32),
            pltpu.VMEM((SEDGE, 16), f32),
            pltpu.VMEM_SHARED((N, HROW), f32),
            pltpu.SemaphoreType.DMA((2,)),
        ],
        compiler_params=_SC_PARAMS,
    )
    return f(vab, exx, src, dst, zeros80)


def _full(shape):
    return pl.BlockSpec(shape, lambda idx: tuple(0 for _ in shape))


def _embed_call(t_pad, kap, W1, b1, W2, b2, WeT, be, seg2d, res2d,
                WA1, WA2, WB1, WB2, ones80):
    nbs = lambda idx: (idx, 0)
    return pl.pallas_call(
        _embed_body,
        grid=(N // NB,),
        in_specs=[
            _full((56, 1)), _full((1, HT // 2)), _full((HT, 128)),
            _full((1, 128)), _full((128, HT)), _full((1, HT)),
            _full((HT, C)), _full((1, C)),
            pl.BlockSpec((NB, 1), nbs),
            pl.BlockSpec((NB, L * C), nbs),
            _full((L * C, HROW)), _full((L * C, HROW)),
            _full((C, HROW)), _full((C, HROW)), _full((1, HROW)),
        ],
        out_specs=[
            pl.BlockSpec((NB, C), nbs),
            pl.BlockSpec((2, NB, HROW), lambda idx: (0, idx, 0)),
        ],
        out_shape=[
            jax.ShapeDtypeStruct((N, C), f32),
            jax.ShapeDtypeStruct((2, N, HROW), f32),
        ],
    )(t_pad, kap, W1, b1, W2, b2, WeT, be, seg2d, res2d,
      WA1, WA2, WB1, WB2, ones80)


def _e1_call(gsout, ef, Wq, Wk, Wea, S16s):
    return pl.pallas_call(
        _e1_body,
        grid=(E // EB,),
        in_specs=[
            pl.BlockSpec((EB, 2 * C), lambda idx: (idx, 0)),
            pl.BlockSpec((EB, 2 * C), lambda idx: (idx + E // EB, 0)),
            pl.BlockSpec((EB, DE), lambda idx: (idx, 0)),
            _full((2 * C, H * DA)), _full((2 * C, H * DA)),
            _full((DE, 16)), _full((H * DA, 16)),
        ],
        out_specs=pl.BlockSpec((EB, 16), lambda idx: (idx, 0)),
        out_shape=jax.ShapeDtypeStruct((E, 16), f32),
    )(gsout, gsout, ef, Wq, Wk, Wea, S16s)


def _ef_call(ef, gsout, WeA, WeB, WeC, Wq, Wk, Wea, S16s):
    return pl.pallas_call(
        _ef_body,
        grid=(E // EB,),
        in_specs=[
            pl.BlockSpec((EB, DE), lambda idx: (idx, 0)),
            pl.BlockSpec((EB, 2 * C), lambda idx: (idx, 0)),
            pl.BlockSpec((EB, 2 * C), lambda idx: (idx + E // EB, 0)),
            _full((DE, DE)), _full((2 * C, DE)), _full((2 * C, DE)),
            _full((2 * C, H * DA)), _full((2 * C, H * DA)),
            _full((DE, 16)), _full((H * DA, 16)),
        ],
        out_specs=[
            pl.BlockSpec((EB, DE), lambda idx: (idx, 0)),
            pl.BlockSpec((EB, 16), lambda idx: (idx, 0)),
        ],
        out_shape=[
            jax.ShapeDtypeStruct((E, DE), f32),
            jax.ShapeDtypeStruct((E, 16), f32),
        ],
    )(ef, gsout, gsout, WeA, WeB, WeC, Wq, Wk, Wea, S16s)


def _n2_call(out2, node2d, res2d, SELA, SELB, WoA, WoB, G32, Hx,
             gam, W1fBD, T64, W2fBD, WrA, WrB, VA1, VA2, VB1, VB2, ones80):
    nbs = lambda idx: (idx, 0)
    return pl.pallas_call(
        _n2_body,
        grid=(N // NB,),
        in_specs=[
            pl.BlockSpec((NB, HROW), nbs),
            pl.BlockSpec((NB, HROW), lambda idx: (idx + N // NB, 0)),
            pl.BlockSpec((NB, L * C), nbs), pl.BlockSpec((NB, L * C), nbs),
            _full((HROW, HROW)), _full((HROW, HROW)),
            _full((HROW, L * C)), _full((HROW, L * C)),
            _full((L * C, L)), _full((L, L * C)), _full((1, L * C)),
            _full((L * C, L * 2 * C)), _full((2 * C, L * 2 * C)),
            _full((L * 2 * C, L * C)),
            _full((L * C, L * C)), _full((L * C, L * C)),
            _full((L * C, HROW)), _full((L * C, HROW)),
            _full((L * C, HROW)), _full((L * C, HROW)), _full((1, HROW)),
        ],
        out_specs=[
            pl.BlockSpec((NB, L * C), nbs), pl.BlockSpec((NB, L * C), nbs),
            pl.BlockSpec((NB, C), nbs), pl.BlockSpec((NB, C), nbs),
            pl.BlockSpec((2, NB, HROW), lambda idx: (0, idx, 0)),
        ],
        out_shape=[
            jax.ShapeDtypeStruct((N, L * C), f32),
            jax.ShapeDtypeStruct((N, L * C), f32),
            jax.ShapeDtypeStruct((N, C), f32),
            jax.ShapeDtypeStruct((N, C), f32),
            jax.ShapeDtypeStruct((2, N, HROW), f32),
        ],
    )(out2, out2, node2d, res2d, SELA, SELB, WoA, WoB, G32, Hx, gam,
      W1fBD, T64, W2fBD, WrA, WrB, VA1, VA2, VB1, VB2, ones80)


def kernel(res_emb, t, edge_s, kappa, W1, b1, W2, b2, W_emb, b_emb, Wq, Wk,
           We_attn, Wv, Wo, gamma, W1f, W2f, Wres, We1, seg_ids, edge_index):
    res2d = res_emb.reshape(N, L * C).astype(f32)
    src = edge_index[0].astype(i32).reshape(NCH, CHUNK)
    dst = edge_index[1].astype(i32).reshape(NCH, CHUNK)
    srcdst = jnp.concatenate([src, dst], axis=0)
    seg2d = seg_ids.reshape(N, 1).astype(i32)
    t_pad = jnp.pad(t.astype(f32), (0, 6)).reshape(56, 1)
    kap = kappa.reshape(1, HT // 2).astype(f32)

    eye9 = jnp.eye(L, dtype=f32)

    def bd9(A):
        return jnp.kron(eye9, A.astype(f32))

    _r2 = np.kron(np.eye(H), np.ones((1, DV)))
    S16s = jnp.asarray(
        np.kron(np.eye(H), np.ones((DA, 1))) @ _r2 / np.sqrt(DA), f32)
    G32 = jnp.asarray(np.kron(np.eye(L), np.ones((C, 1))) / C, f32)
    Hx = jnp.asarray(np.kron(np.eye(L), np.ones((1, C))), f32)
    T64 = jnp.asarray(np.kron(np.ones((1, L)), np.eye(2 * C)), f32)
    selA = np.zeros((HROW, HROW), np.float32)
    selB = np.zeros((HROW, HROW), np.float32)
    for tt in range(HROW):
        selA[64 + tt % 16, tt] = 1.0
    for tt in range(64):
        selB[64 + tt % 16, tt] = 1.0
    for tt in range(64, HROW):
        selB[tt, tt] = 1.0
    SELA = jnp.asarray(selA, f32)
    SELB = jnp.asarray(selB, f32)
    ones80 = jnp.concatenate(
        [jnp.zeros((1, 64), f32), jnp.ones((1, 16), f32)], axis=1)
    zeros80 = jnp.zeros((N, HROW), f32)
    zpad = jnp.zeros((C, DE), f32)

    def vsplit(i):
        A = bd9(Wv[i][:C])
        Bm = bd9(Wv[i][C:])
        z = jnp.zeros((L * C, 16), f32)
        return (A[:, :HROW], jnp.concatenate([A[:, HROW:], z], 1),
                Bm[:, :HROW], jnp.concatenate([Bm[:, HROW:], z], 1))

    def wo_split(i):
        W = bd9(Wo[i])
        return W[:HROW], jnp.concatenate(
            [W[HROW:], jnp.zeros((16, L * C), f32)], 0)

    VA1, VA2, VB1, VB2 = vsplit(0)
    l0, v2 = _embed_call(t_pad, kap, W1.astype(f32), b1.reshape(1, 128),
                         W2.astype(f32), b2.reshape(1, HT),
                         W_emb[C:].astype(f32), b_emb.reshape(1, C), seg2d,
                         res2d, VA1, VA2, VB1[:C], VB2[:C], ones80)
    node2d = jnp.concatenate([l0, jnp.zeros((N, (L - 1) * C), f32)], axis=1)
    s = jnp.concatenate([res2d[:, :C], l0], axis=1)
    ef = edge_s.astype(f32)

    R2x = jnp.asarray(_r2, f32)
    gsout = _gs_call(s.astype(jnp.bfloat16), srcdst)
    exx = _e1_call(gsout, ef, Wq[0], Wk[0], We_attn[0] @ R2x, S16s)

    for i in range(NL):
        vab = v2.reshape(2 * N, HROW)
        out2 = _agg_call(vab, exx, src, dst, zeros80)
        WoA, WoB = wo_split(i)
        VA1, VA2, VB1, VB2 = vsplit(min(i + 1, NL - 1))
        res2d, node2d, sa, sb, v2 = _n2_call(
            out2, node2d, res2d, SELA, SELB, WoA, WoB, G32, Hx,
            jnp.tile(gamma[i].reshape(1, C), (1, L)),
            bd9(W1f[i]), T64, bd9(W2f[i]), bd9(Wres[i][:C]),
            bd9(Wres[i][C:]), VA1, VA2, VB1, VB2, ones80)
        s = jnp.concatenate([sa, sb], axis=1)
        if i < NL - 1:
            gsout = _gs_call(s.astype(jnp.bfloat16), srcdst)
            ef, exx = _ef_call(
                ef, gsout, We1[i][:DE],
                jnp.concatenate([zpad, We1[i][DE:DE + C]], axis=0),
                jnp.concatenate([zpad, We1[i][DE + C:]], axis=0),
                Wq[i + 1], Wk[i + 1], We_attn[i + 1] @ R2x, S16s)

    return res2d.reshape(N, L, C)


# speedup vs baseline: 1.0718x; 1.0718x over previous
import functools

import jax
import jax.numpy as jnp
import numpy as np
from jax.experimental import pallas as pl
from jax.experimental.pallas import tpu as pltpu
from jax.experimental.pallas import tpu_sc as plsc

N = 10000
E = 160000
B = 50
C = 32
L = 9
H = 8
DA = 16
DV = 2
NL = 4
DE = 64
HT = 64

f32 = jnp.float32
i32 = jnp.int32

NCORES = 2
NSUB = 16
NTILES = NCORES * NSUB
CHUNK = 128
NCH = E // CHUNK
SUPER = 2
SEDGE = SUPER * CHUNK
NSS = E // SEDGE
KAGG = (NSS + NSUB - 1) // NSUB
NGS = 2 * NSS
KGS = (NGS + NTILES - 1) // NTILES
STRIPE = N // NSUB
HROW = 80

NB = 2000
EB = 8000


def _dot(a, b):
    return jnp.dot(a, b, preferred_element_type=f32)


def _embed_body(t_ref, kap_ref, W1_ref, b1_ref, W2_ref, b2_ref, We_ref,
                be_ref, seg_ref, res_ref, WA1_ref, WA2_ref, WB1_ref,
                WB2_ref, ones_ref, l0_ref, v2_ref):
    tp = (2.0 * np.pi) * t_ref[:, :] * kap_ref[:, :]
    four = jnp.concatenate([jnp.cos(tp), jnp.sin(tp)], axis=1)
    h1 = jnp.maximum(_dot(four, W1_ref[:, :]) + b1_ref[:, :], 0.0)
    et = jnp.maximum(_dot(h1, W2_ref[:, :]) + b2_ref[:, :], 0.0)
    seg = seg_ref[:, :]
    oh = (seg == jax.lax.broadcasted_iota(i32, (seg.shape[0], 56), 1)
          ).astype(f32)
    tn = _dot(oh, et)
    l0 = _dot(tn, We_ref[:, :]) + be_ref[:, :]
    l0_ref[:, :] = l0
    res = res_ref[:, :]
    v2_ref[0, :, :] = _dot(res, WA1_ref[:, :]) + _dot(l0, WB1_ref[:, :])
    v2_ref[1, :, :] = (_dot(res, WA2_ref[:, :]) + _dot(l0, WB2_ref[:, :])
                       + ones_ref[:, :])


def _e1_body(ssrc_ref, sdst_ref, ef_ref, Wq_ref, Wk_ref, Wea_ref, S_ref,
             out_ref):
    q = _dot(sdst_ref[:, :], Wq_ref[:, :])
    k = _dot(ssrc_ref[:, :], Wk_ref[:, :])
    lg = _dot(q * k, S_ref[:, :]) + _dot(ef_ref[:, :], Wea_ref[:, :])
    out_ref[:, :] = jnp.exp(lg)


def _ef_body(ef_ref, ssrc_ref, sdst_ref, WeA_ref, WeB_ref, WeC_ref,
             Wq_ref, Wk_ref, Wea_ref, S_ref, efo_ref, exx_ref):
    ssrc = ssrc_ref[:, :]
    sdst = sdst_ref[:, :]
    efn = jnp.maximum(_dot(ef_ref[:, :], WeA_ref[:, :])
                      + _dot(ssrc, WeB_ref[:, :])
                      + _dot(sdst, WeC_ref[:, :]), 0.0)
    efo_ref[:, :] = efn
    q = _dot(sdst, Wq_ref[:, :])
    k = _dot(ssrc, Wk_ref[:, :])
    lg = _dot(q * k, S_ref[:, :]) + _dot(efn, Wea_ref[:, :])
    exx_ref[:, :] = jnp.exp(lg)


def _n2_body(numA_ref, numB_ref, node_ref, res_ref,
             SELA_ref, SELB_ref, WoA_ref, WoB_ref, G32_ref, Hx_ref,
             gam_ref, W1f_ref, T64_ref, W2f_ref, WrA_ref, WrB_ref,
             VA1_ref, VA2_ref, VB1_ref, VB2_ref, ones_ref,
             rout_ref, nout_ref, sa_ref, sb_ref, v2_ref):
    numA = numA_ref[:, :]
    numB = numB_ref[:, :]
    dentA = _dot(numB, SELA_ref[:, :]) + 1e-9
    dentB = _dot(numB, SELB_ref[:, :]) + 1e-9
    x = (node_ref[:, :] + _dot(numA / dentA, WoA_ref[:, :])
         + _dot(numB / dentB, WoB_ref[:, :]))
    ms = _dot(x * x, G32_ref[:, :])
    rinv = jax.lax.rsqrt(ms + 1e-6)
    xn = x * _dot(rinv, Hx_ref[:, :]) * gam_ref[:, :]
    h = _dot(xn, W1f_ref[:, :])
    g = jax.nn.sigmoid(h[:, :2 * C])
    hg = h * _dot(g, T64_ref[:, :])
    nn = xn + _dot(hg, W2f_ref[:, :])
    res = res_ref[:, :]
    rnew = res + _dot(res, WrA_ref[:, :]) + _dot(nn, WrB_ref[:, :])
    rout_ref[:, :] = rnew
    nout_ref[:, :] = nn
    sa_ref[:, :] = rnew[:, :C]
    sb_ref[:, :] = nn[:, :C]
    v2_ref[0, :, :] = _dot(rnew, VA1_ref[:, :]) + _dot(nn, VB1_ref[:, :])
    v2_ref[1, :, :] = (_dot(rnew, VA2_ref[:, :]) + _dot(nn, VB2_ref[:, :])
                       + ones_ref[:, :])


def _gs_body(s_hbm, sd_hbm, out_hbm, ib, db, sem):
    c = jax.lax.axis_index("c")
    sub = jax.lax.axis_index("s")
    w = sub * NCORES + c

    def fire(k, b):
        tid = k * NTILES + w

        @pl.when(tid < NGS)
        def _():
            pltpu.sync_copy(sd_hbm.at[pl.ds(tid * SUPER, SUPER)], ib.at[b])
            for j in range(SUPER):
                pltpu.async_copy(
                    s_hbm.at[ib.at[b].at[j]],
                    db.at[b].at[pl.ds(j * CHUNK, CHUNK)], sem.at[b])

    def drain(k, b):
        tid = k * NTILES + w

        @pl.when(tid < NGS)
        def _():
            for j in range(SUPER):
                pltpu.make_async_copy(
                    s_hbm.at[ib.at[b].at[j]],
                    db.at[b].at[pl.ds(j * CHUNK, CHUNK)], sem.at[b]).wait()
            pltpu.sync_copy(db.at[b], out_hbm.at[pl.ds(tid * SEDGE, SEDGE)])

    def pipe(m, carry):
        b = jax.lax.rem(m, 2)
        fire(m, b)

        @pl.when(m >= 1)
        def _():
            drain(m - 1, 1 - b)
        return carry

    jax.lax.fori_loop(0, KGS + 1, pipe, 0)


def _agg_body(v_hbm, exx_hbm, src_hbm, dst_hbm, zero_hbm, out_hbm,
              isb, idb, rb, eb, acc, sem):
    c = jax.lax.axis_index("c")
    sub = jax.lax.axis_index("s")

    pltpu.sync_copy(zero_hbm.at[pl.ds(sub * STRIPE, STRIPE)],
                    acc.at[pl.ds(sub * STRIPE, STRIPE)])
    plsc.subcore_barrier()
    base = c * N

    def fire(k, b):
        sid = k * NSUB + sub

        @pl.when(sid < NSS)
        def _():
            cid0 = sid * SUPER
            pltpu.sync_copy(src_hbm.at[pl.ds(cid0, SUPER)], isb.at[b])
            pltpu.sync_copy(dst_hbm.at[pl.ds(cid0, SUPER)], idb.at[b])
            for j in range(SUPER):
                for q in range(CHUNK // 16):
                    sl = pl.ds(q * 16, 16)
                    isb[b, j, sl] = isb[b, j, sl] + base
                pltpu.async_copy(
                    v_hbm.at[isb.at[b].at[j]],
                    rb.at[b].at[pl.ds(j * CHUNK, CHUNK)], sem.at[b])

    def process(k, b):
        sid = k * NSUB + sub

        @pl.when(sid < NSS)
        def _():
            pltpu.sync_copy(exx_hbm.at[pl.ds(sid * SEDGE, SEDGE)], eb)
            for j in range(SUPER):
                pltpu.make_async_copy(
                    v_hbm.at[isb.at[b].at[j]],
                    rb.at[b].at[pl.ds(j * CHUNK, CHUNK)], sem.at[b]).wait()

            def estep(e, cy):
                exr = eb[e, :]
                for kk in range(HROW // 16):
                    sl = pl.ds(kk * 16, 16)
                    rb[b, e, sl] = rb[b, e, sl] * exr
                return cy

            jax.lax.fori_loop(0, SEDGE, estep, 0)
            for j in range(SUPER):
                pltpu.sync_copy(rb.at[b].at[pl.ds(j * CHUNK, CHUNK)],
                                acc.at[idb.at[b].at[j]], add=True)

    def pipe(m, carry):
        b = jax.lax.rem(m, 2)
        fire(m, b)

        @pl.when(m >= 1)
        def _():
            process(m - 1, 1 - b)
        return carry

    jax.lax.fori_loop(0, KAGG + 1, pipe, 0)

    plsc.subcore_barrier()
    pltpu.sync_copy(acc.at[pl.ds(sub * STRIPE, STRIPE)],
                    out_hbm.at[pl.ds(c * N + sub * STRIPE, STRIPE)])


def _mesh():
    return plsc.VectorSubcoreMesh(core_axis_name="c", subcore_axis_name="s")


_SC_PARAMS = pltpu.CompilerParams(use_tc_tiling_on_sc=False)


def _gs_call(s, srcdst):
    f = pl.kernel(
        _gs_body,
        out_type=jax.ShapeDtypeStruct((2 * E, 2 * C), f32),
        mesh=_mesh(),
        scratch_types=[
            pltpu.VMEM((2, SUPER, CHUNK), i32),
            pltpu.VMEM((2, SEDGE, 2 * C), f32),
            pltpu.SemaphoreType.DMA((2,)),
        ],
        compiler_params=_SC_PARAMS,
    )
    return f(s, srcdst)


def _agg_call(vab, exx, src, dst, zeros80):
    f = pl.kernel(
        _agg_body,
        out_type=jax.ShapeDtypeStruct((2 * N, HROW), f32),
        mesh=_mesh(),
        scratch_types=[
            pltpu.VMEM((2, SUPER, CHUNK), i32),
            pltpu.VMEM((2, SUPER, CHUNK), i32),
            pltpu.VMEM((2, SEDGE, HROW), f32),
            pltpu.VMEM((SEDGE, 16), f32),
            pltpu.VMEM_SHARED((N, HROW), f32),
            pltpu.SemaphoreType.DMA((2,)),
        ],
        compiler_params=_SC_PARAMS,
    )
    return f(vab, exx, src, dst, zeros80)


def _full(shape):
    return pl.BlockSpec(shape, lambda idx: tuple(0 for _ in shape))


def _embed_call(t_pad, kap, W1, b1, W2, b2, WeT, be, seg2d, res2d,
                WA1, WA2, WB1, WB2, ones80):
    nbs = lambda idx: (idx, 0)
    return pl.pallas_call(
        _embed_body,
        grid=(N // NB,),
        in_specs=[
            _full((56, 1)), _full((1, HT // 2)), _full((HT, 128)),
            _full((1, 128)), _full((128, HT)), _full((1, HT)),
            _full((HT, C)), _full((1, C)),
            pl.BlockSpec((NB, 1), nbs),
            pl.BlockSpec((NB, L * C), nbs),
            _full((L * C, HROW)), _full((L * C, HROW)),
            _full((C, HROW)), _full((C, HROW)), _full((1, HROW)),
        ],
        out_specs=[
            pl.BlockSpec((NB, C), nbs),
            pl.BlockSpec((2, NB, HROW), lambda idx: (0, idx, 0)),
        ],
        out_shape=[
            jax.ShapeDtypeStruct((N, C), f32),
            jax.ShapeDtypeStruct((2, N, HROW), f32),
        ],
    )(t_pad, kap, W1, b1, W2, b2, WeT, be, seg2d, res2d,
      WA1, WA2, WB1, WB2, ones80)


def _e1_call(gsout, ef, Wq, Wk, Wea, S16s):
    return pl.pallas_call(
        _e1_body,
        grid=(E // EB,),
        in_specs=[
            pl.BlockSpec((EB, 2 * C), lambda idx: (idx, 0)),
            pl.BlockSpec((EB, 2 * C), lambda idx: (idx + E // EB, 0)),
            pl.BlockSpec((EB, DE), lambda idx: (idx, 0)),
            _full((2 * C, H * DA)), _full((2 * C, H * DA)),
            _full((DE, 16)), _full((H * DA, 16)),
        ],
        out_specs=pl.BlockSpec((EB, 16), lambda idx: (idx, 0)),
        out_shape=jax.ShapeDtypeStruct((E, 16), f32),
    )(gsout, gsout, ef, Wq, Wk, Wea, S16s)


def _ef_call(ef, gsout, WeA, WeB, WeC, Wq, Wk, Wea, S16s):
    return pl.pallas_call(
        _ef_body,
        grid=(E // EB,),
        in_specs=[
            pl.BlockSpec((EB, DE), lambda idx: (idx, 0)),
            pl.BlockSpec((EB, 2 * C), lambda idx: (idx, 0)),
            pl.BlockSpec((EB, 2 * C), lambda idx: (idx + E // EB, 0)),
            _full((DE, DE)), _full((2 * C, DE)), _full((2 * C, DE)),
            _full((2 * C, H * DA)), _full((2 * C, H * DA)),
            _full((DE, 16)), _full((H * DA, 16)),
        ],
        out_specs=[
            pl.BlockSpec((EB, DE), lambda idx: (idx, 0)),
            pl.BlockSpec((EB, 16), lambda idx: (idx, 0)),
        ],
        out_shape=[
            jax.ShapeDtypeStruct((E, DE), f32),
            jax.ShapeDtypeStruct((E, 16), f32),
        ],
    )(ef, gsout, gsout, WeA, WeB, WeC, Wq, Wk, Wea, S16s)


def _n2_call(out2, node2d, res2d, SELA, SELB, WoA, WoB, G32, Hx,
             gam, W1fBD, T64, W2fBD, WrA, WrB, VA1, VA2, VB1, VB2, ones80):
    nbs = lambda idx: (idx, 0)
    return pl.pallas_call(
        _n2_body,
        grid=(N // NB,),
        in_specs=[
            pl.BlockSpec((NB, HROW), nbs),
            pl.BlockSpec((NB, HROW), lambda idx: (idx + N // NB, 0)),
            pl.BlockSpec((NB, L * C), nbs), pl.BlockSpec((NB, L * C), nbs),
            _full((HROW, HROW)), _full((HROW, HROW)),
            _full((HROW, L * C)), _full((HROW, L * C)),
            _full((L * C, L)), _full((L, L * C)), _full((1, L * C)),
            _full((L * C, L * 2 * C)), _full((2 * C, L * 2 * C)),
            _full((L * 2 * C, L * C)),
            _full((L * C, L * C)), _full((L * C, L * C)),
            _full((L * C, HROW)), _full((L * C, HROW)),
            _full((L * C, HROW)), _full((L * C, HROW)), _full((1, HROW)),
        ],
        out_specs=[
            pl.BlockSpec((NB, L * C), nbs), pl.BlockSpec((NB, L * C), nbs),
            pl.BlockSpec((NB, C), nbs), pl.BlockSpec((NB, C), nbs),
            pl.BlockSpec((2, NB, HROW), lambda idx: (0, idx, 0)),
        ],
        out_shape=[
            jax.ShapeDtypeStruct((N, L * C), f32),
            jax.ShapeDtypeStruct((N, L * C), f32),
            jax.ShapeDtypeStruct((N, C), f32),
            jax.ShapeDtypeStruct((N, C), f32),
            jax.ShapeDtypeStruct((2, N, HROW), f32),
        ],
    )(out2, out2, node2d, res2d, SELA, SELB, WoA, WoB, G32, Hx, gam,
      W1fBD, T64, W2fBD, WrA, WrB, VA1, VA2, VB1, VB2, ones80)


def kernel(res_emb, t, edge_s, kappa, W1, b1, W2, b2, W_emb, b_emb, Wq, Wk,
           We_attn, Wv, Wo, gamma, W1f, W2f, Wres, We1, seg_ids, edge_index):
    res2d = res_emb.reshape(N, L * C).astype(f32)
    src = edge_index[0].astype(i32).reshape(NCH, CHUNK)
    dst = edge_index[1].astype(i32).reshape(NCH, CHUNK)
    srcdst = jnp.concatenate([src, dst], axis=0)
    seg2d = seg_ids.reshape(N, 1).astype(i32)
    t_pad = jnp.pad(t.astype(f32), (0, 6)).reshape(56, 1)
    kap = kappa.reshape(1, HT // 2).astype(f32)

    eye9 = jnp.eye(L, dtype=f32)

    def bd9(A):
        return jnp.kron(eye9, A.astype(f32))

    _r2 = np.kron(np.eye(H), np.ones((1, DV)))
    S16s = jnp.asarray(
        np.kron(np.eye(H), np.ones((DA, 1))) @ _r2 / np.sqrt(DA), f32)
    G32 = jnp.asarray(np.kron(np.eye(L), np.ones((C, 1))) / C, f32)
    Hx = jnp.asarray(np.kron(np.eye(L), np.ones((1, C))), f32)
    T64 = jnp.asarray(np.kron(np.ones((1, L)), np.eye(2 * C)), f32)
    selA = np.zeros((HROW, HROW), np.float32)
    selB = np.zeros((HROW, HROW), np.float32)
    for tt in range(HROW):
        selA[64 + tt % 16, tt] = 1.0
    for tt in range(64):
        selB[64 + tt % 16, tt] = 1.0
    for tt in range(64, HROW):
        selB[tt, tt] = 1.0
    SELA = jnp.asarray(selA, f32)
    SELB = jnp.asarray(selB, f32)
    ones80 = jnp.concatenate(
        [jnp.zeros((1, 64), f32), jnp.ones((1, 16), f32)], axis=1)
    zeros80 = jnp.zeros((N, HROW), f32)
    zpad = jnp.zeros((C, DE), f32)

    def vsplit(i):
        A = bd9(Wv[i][:C])
        Bm = bd9(Wv[i][C:])
        z = jnp.zeros((L * C, 16), f32)
        return (A[:, :HROW], jnp.concatenate([A[:, HROW:], z], 1),
                Bm[:, :HROW], jnp.concatenate([Bm[:, HROW:], z], 1))

    def wo_split(i):
        W = bd9(Wo[i])
        return W[:HROW], jnp.concatenate(
            [W[HROW:], jnp.zeros((16, L * C), f32)], 0)

    VA1, VA2, VB1, VB2 = vsplit(0)
    l0, v2 = _embed_call(t_pad, kap, W1.astype(f32), b1.reshape(1, 128),
                         W2.astype(f32), b2.reshape(1, HT),
                         W_emb[C:].astype(f32), b_emb.reshape(1, C), seg2d,
                         res2d, VA1, VA2, VB1[:C], VB2[:C], ones80)
    node2d = jnp.concatenate([l0, jnp.zeros((N, (L - 1) * C), f32)], axis=1)
    s = jnp.concatenate([res2d[:, :C], l0], axis=1)
    ef = edge_s.astype(f32)

    R2x = jnp.asarray(_r2, f32)
    gsout = _gs_call(s, srcdst)
    exx = _e1_call(gsout, ef, Wq[0], Wk[0], We_attn[0] @ R2x, S16s)

    for i in range(NL):
        vab = v2.reshape(2 * N, HROW)
        out2 = _agg_call(vab, exx, src, dst, zeros80)
        WoA, WoB = wo_split(i)
        VA1, VA2, VB1, VB2 = vsplit(min(i + 1, NL - 1))
        res2d, node2d, sa, sb, v2 = _n2_call(
            out2, node2d, res2d, SELA, SELB, WoA, WoB, G32, Hx,
            jnp.tile(gamma[i].reshape(1, C), (1, L)),
            bd9(W1f[i]), T64, bd9(W2f[i]), bd9(Wres[i][:C]),
            bd9(Wres[i][C:]), VA1, VA2, VB1, VB2, ones80)
        s = jnp.concatenate([sa, sb], axis=1)
        if i < NL - 1:
            gsout = _gs_call(s, srcdst)
            ef, exx = _ef_call(
                ef, gsout, We1[i][:DE],
                jnp.concatenate([zpad, We1[i][DE:DE + C]], axis=0),
                jnp.concatenate([zpad, We1[i][DE + C:]], axis=0),
                Wq[i + 1], Wk[i + 1], We_attn[i + 1] @ R2x, S16s)

    return res2d.reshape(N, L, C)


# speedup vs baseline: 1.0724x; 1.0005x over previous
import jax
import jax.numpy as jnp
import numpy as np
from jax.experimental import pallas as pl
from jax.experimental.pallas import tpu as pltpu
from jax.experimental.pallas import tpu_sc as plsc

N = 10000
E = 160000
B = 50
C = 32
L = 9
H = 8
DA = 16
DV = 2
NL = 4
DE = 64
HT = 64

f32 = jnp.float32
i32 = jnp.int32

NCORES = 2
NSUB = 16
NTILES = NCORES * NSUB
CHUNK = 128
NCH = E // CHUNK
SUPER = 2
SEDGE = SUPER * CHUNK
NSS = E // SEDGE
KAGG = (NSS + NSUB - 1) // NSUB
NGS = 2 * NSS
KGS = (NGS + NTILES - 1) // NTILES
STRIPE = N // NSUB
HROW = 80

NB = 2000
EB = 8000


def _dot(a, b):
    return jnp.dot(a, b, preferred_element_type=f32)


def _embed_body(t_ref, kap_ref, W1_ref, b1_ref, W2_ref, b2_ref, We_ref,
                be_ref, seg_ref, res_ref, WA1_ref, WA2_ref, WB1_ref,
                WB2_ref, ones_ref, l0_ref, v2_ref):
    tp = (2.0 * np.pi) * t_ref[:, :] * kap_ref[:, :]
    four = jnp.concatenate([jnp.cos(tp), jnp.sin(tp)], axis=1)
    h1 = jnp.maximum(_dot(four, W1_ref[:, :]) + b1_ref[:, :], 0.0)
    et = jnp.maximum(_dot(h1, W2_ref[:, :]) + b2_ref[:, :], 0.0)
    seg = seg_ref[:, :]
    oh = (seg == jax.lax.broadcasted_iota(i32, (seg.shape[0], 56), 1)
          ).astype(f32)
    tn = _dot(oh, et)
    l0 = _dot(tn, We_ref[:, :]) + be_ref[:, :]
    l0_ref[:, :] = l0
    res = res_ref[:, :]
    v2_ref[0, :, :] = _dot(res, WA1_ref[:, :]) + _dot(l0, WB1_ref[:, :])
    v2_ref[1, :, :] = (_dot(res, WA2_ref[:, :]) + _dot(l0, WB2_ref[:, :])
                       + ones_ref[:, :])


def _e1_body(ssrc_ref, sdst_ref, ef_ref, Wq_ref, Wk_ref, Wea_ref, S_ref,
             out_ref):
    q = _dot(sdst_ref[:, :], Wq_ref[:, :])
    k = _dot(ssrc_ref[:, :], Wk_ref[:, :])
    lg = _dot(q * k, S_ref[:, :]) + _dot(ef_ref[:, :], Wea_ref[:, :])
    out_ref[:, :] = jnp.exp(lg)


def _ef_body(ef_ref, ssrc_ref, sdst_ref, WeA_ref, WeB_ref, WeC_ref,
             Wq_ref, Wk_ref, Wea_ref, S_ref, efo_ref, exx_ref):
    ssrc = ssrc_ref[:, :]
    sdst = sdst_ref[:, :]
    efn = jnp.maximum(_dot(ef_ref[:, :], WeA_ref[:, :])
                      + _dot(ssrc, WeB_ref[:, :])
                      + _dot(sdst, WeC_ref[:, :]), 0.0)
    efo_ref[:, :] = efn
    q = _dot(sdst, Wq_ref[:, :])
    k = _dot(ssrc, Wk_ref[:, :])
    lg = _dot(q * k, S_ref[:, :]) + _dot(efn, Wea_ref[:, :])
    exx_ref[:, :] = jnp.exp(lg)


def _n2_body(numA_ref, numB_ref, node_ref, res_ref,
             SELA_ref, SELB_ref, WoA_ref, WoB_ref, G32_ref, Hx_ref,
             gam_ref, W1f_ref, T64_ref, W2f_ref, WrA_ref, WrB_ref,
             VA1_ref, VA2_ref, VB1_ref, VB2_ref, ones_ref,
             rout_ref, nout_ref, sa_ref, sb_ref, v2_ref):
    numA = numA_ref[:, :]
    numB = numB_ref[:, :]
    dentA = _dot(numB, SELA_ref[:, :]) + 1e-9
    dentB = _dot(numB, SELB_ref[:, :]) + 1e-9
    x = (node_ref[:, :] + _dot(numA / dentA, WoA_ref[:, :])
         + _dot(numB / dentB, WoB_ref[:, :]))
    ms = _dot(x * x, G32_ref[:, :])
    rinv = jax.lax.rsqrt(ms + 1e-6)
    xn = x * _dot(rinv, Hx_ref[:, :]) * gam_ref[:, :]
    h = _dot(xn, W1f_ref[:, :])
    g = jax.nn.sigmoid(h[:, :2 * C])
    hg = h * _dot(g, T64_ref[:, :])
    nn = xn + _dot(hg, W2f_ref[:, :])
    res = res_ref[:, :]
    rnew = res + _dot(res, WrA_ref[:, :]) + _dot(nn, WrB_ref[:, :])
    rout_ref[:, :] = rnew
    nout_ref[:, :] = nn
    sa_ref[:, :] = rnew[:, :C]
    sb_ref[:, :] = nn[:, :C]
    v2_ref[0, :, :] = _dot(rnew, VA1_ref[:, :]) + _dot(nn, VB1_ref[:, :])
    v2_ref[1, :, :] = (_dot(rnew, VA2_ref[:, :]) + _dot(nn, VB2_ref[:, :])
                       + ones_ref[:, :])


def _gs_body(s_hbm, sd_hbm, out_hbm, ib, db, sem):
    c = jax.lax.axis_index("c")
    sub = jax.lax.axis_index("s")
    w = sub * NCORES + c

    def fire(k, b):
        tid = k * NTILES + w

        @pl.when(tid < NGS)
        def _():
            pltpu.sync_copy(sd_hbm.at[pl.ds(tid * SUPER, SUPER)], ib.at[b])
            for j in range(SUPER):
                pltpu.async_copy(
                    s_hbm.at[ib.at[b].at[j]],
                    db.at[b].at[pl.ds(j * CHUNK, CHUNK)], sem.at[b])

    def drain(k, b):
        tid = k * NTILES + w

        @pl.when(tid < NGS)
        def _():
            for j in range(SUPER):
                pltpu.make_async_copy(
                    s_hbm.at[ib.at[b].at[j]],
                    db.at[b].at[pl.ds(j * CHUNK, CHUNK)], sem.at[b]).wait()
            pltpu.sync_copy(db.at[b], out_hbm.at[pl.ds(tid * SEDGE, SEDGE)])

    def pipe(m, carry):
        b = jax.lax.rem(m, 2)
        fire(m, b)

        @pl.when(m >= 1)
        def _():
            drain(m - 1, 1 - b)
        return carry

    jax.lax.fori_loop(0, KGS + 1, pipe, 0)


def _agg_body(v_hbm, exx_hbm, src_hbm, dst_hbm, zero_hbm, out_hbm,
              isb, idb, rb, eb, acc, sem):
    c = jax.lax.axis_index("c")
    sub = jax.lax.axis_index("s")

    pltpu.sync_copy(zero_hbm.at[pl.ds(sub * STRIPE, STRIPE)],
                    acc.at[pl.ds(sub * STRIPE, STRIPE)])
    plsc.subcore_barrier()
    base = c * N

    def fire(k, b):
        sid = k * NSUB + sub

        @pl.when(sid < NSS)
        def _():
            cid0 = sid * SUPER
            pltpu.sync_copy(src_hbm.at[pl.ds(cid0, SUPER)], isb.at[b])
            pltpu.sync_copy(dst_hbm.at[pl.ds(cid0, SUPER)], idb.at[b])
            for j in range(SUPER):
                for q in range(CHUNK // 16):
                    sl = pl.ds(q * 16, 16)
                    isb[b, j, sl] = isb[b, j, sl] + base
                pltpu.async_copy(
                    v_hbm.at[isb.at[b].at[j]],
                    rb.at[b].at[pl.ds(j * CHUNK, CHUNK)], sem.at[b])

    def process(k, b):
        sid = k * NSUB + sub

        @pl.when(sid < NSS)
        def _():
            pltpu.sync_copy(exx_hbm.at[pl.ds(sid * SEDGE, SEDGE)], eb)
            for j in range(SUPER):
                pltpu.make_async_copy(
                    v_hbm.at[isb.at[b].at[j]],
                    rb.at[b].at[pl.ds(j * CHUNK, CHUNK)], sem.at[b]).wait()

            def estep(e, cy):
                exr = eb[e, :]
                for kk in range(HROW // 16):
                    sl = pl.ds(kk * 16, 16)
                    rb[b, e, sl] = rb[b, e, sl] * exr
                return cy

            jax.lax.fori_loop(0, SEDGE, estep, 0)
            for j in range(SUPER):
                pltpu.sync_copy(rb.at[b].at[pl.ds(j * CHUNK, CHUNK)],
                                acc.at[idb.at[b].at[j]], add=True)

    def pipe(m, carry):
        b = jax.lax.rem(m, 2)
        fire(m, b)

        @pl.when(m >= 1)
        def _():
            process(m - 1, 1 - b)
        return carry

    jax.lax.fori_loop(0, KAGG + 1, pipe, 0)

    plsc.subcore_barrier()
    pltpu.sync_copy(acc.at[pl.ds(sub * STRIPE, STRIPE)],
                    out_hbm.at[pl.ds(c * N + sub * STRIPE, STRIPE)])


def _mesh():
    return plsc.VectorSubcoreMesh(core_axis_name="c", subcore_axis_name="s")


_SC_PARAMS = pltpu.CompilerParams(use_tc_tiling_on_sc=False)


def _gs_call(s, srcdst):
    f = pl.kernel(
        _gs_body,
        out_type=jax.ShapeDtypeStruct((2 * E, 2 * C), f32),
        mesh=_mesh(),
        scratch_types=[
            pltpu.VMEM((2, SUPER, CHUNK), i32),
            pltpu.VMEM((2, SEDGE, 2 * C), f32),
            pltpu.SemaphoreType.DMA((2,)),
        ],
        compiler_params=_SC_PARAMS,
    )
    return f(s, srcdst)


def _agg_call(vab, exx, src, dst, zeros80):
    f = pl.kernel(
        _agg_body,
        out_type=jax.ShapeDtypeStruct((2 * N, HROW), f32),
        mesh=_mesh(),
        scratch_types=[
            pltpu.VMEM((2, SUPER, CHUNK), i32),
            pltpu.VMEM((2, SUPER, CHUNK), i32),
            pltpu.VMEM((2, SEDGE, HROW), f32),
            pltpu.VMEM((SEDGE, 16), f32),
            pltpu.VMEM_SHARED((N, HROW), f32),
            pltpu.SemaphoreType.DMA((2,)),
        ],
        compiler_params=_SC_PARAMS,
    )
    return f(vab, exx, src, dst, zeros80)


def _full(shape):
    return pl.BlockSpec(shape, lambda idx: tuple(0 for _ in shape))


def _embed_call(t_pad, kap, W1, b1, W2, b2, WeT, be, seg2d, res2d,
                WA1, WA2, WB1, WB2, ones80):
    nbs = lambda idx: (idx, 0)
    return pl.pallas_call(
        _embed_body,
        grid=(N // NB,),
        in_specs=[
            _full((56, 1)), _full((1, HT // 2)), _full((HT, 128)),
            _full((1, 128)), _full((128, HT)), _full((1, HT)),
            _full((HT, C)), _full((1, C)),
            pl.BlockSpec((NB, 1), nbs),
            pl.BlockSpec((NB, L * C), nbs),
            _full((L * C, HROW)), _full((L * C, HROW)),
            _full((C, HROW)), _full((C, HROW)), _full((1, HROW)),
        ],
        out_specs=[
            pl.BlockSpec((NB, C), nbs),
            pl.BlockSpec((2, NB, HROW), lambda idx: (0, idx, 0)),
        ],
        out_shape=[
            jax.ShapeDtypeStruct((N, C), f32),
            jax.ShapeDtypeStruct((2, N, HROW), f32),
        ],
    )(t_pad, kap, W1, b1, W2, b2, WeT, be, seg2d, res2d,
      WA1, WA2, WB1, WB2, ones80)


def _e1_call(gsout, ef, Wq, Wk, Wea, S16s):
    return pl.pallas_call(
        _e1_body,
        grid=(E // EB,),
        in_specs=[
            pl.BlockSpec((EB, 2 * C), lambda idx: (idx, 0)),
            pl.BlockSpec((EB, 2 * C), lambda idx: (idx + E // EB, 0)),
            pl.BlockSpec((EB, DE), lambda idx: (idx, 0)),
            _full((2 * C, H * DA)), _full((2 * C, H * DA)),
            _full((DE, 16)), _full((H * DA, 16)),
        ],
        out_specs=pl.BlockSpec((EB, 16), lambda idx: (idx, 0)),
        out_shape=jax.ShapeDtypeStruct((E, 16), f32),
    )(gsout, gsout, ef, Wq, Wk, Wea, S16s)


def _ef_call(ef, gsout, WeA, WeB, WeC, Wq, Wk, Wea, S16s):
    return pl.pallas_call(
        _ef_body,
        grid=(E // EB,),
        in_specs=[
            pl.BlockSpec((EB, DE), lambda idx: (idx, 0)),
            pl.BlockSpec((EB, 2 * C), lambda idx: (idx, 0)),
            pl.BlockSpec((EB, 2 * C), lambda idx: (idx + E // EB, 0)),
            _full((DE, DE)), _full((2 * C, DE)), _full((2 * C, DE)),
            _full((2 * C, H * DA)), _full((2 * C, H * DA)),
            _full((DE, 16)), _full((H * DA, 16)),
        ],
        out_specs=[
            pl.BlockSpec((EB, DE), lambda idx: (idx, 0)),
            pl.BlockSpec((EB, 16), lambda idx: (idx, 0)),
        ],
        out_shape=[
            jax.ShapeDtypeStruct((E, DE), f32),
            jax.ShapeDtypeStruct((E, 16), f32),
        ],
    )(ef, gsout, gsout, WeA, WeB, WeC, Wq, Wk, Wea, S16s)


def _n2_call(out2, node2d, res2d, SELA, SELB, WoA, WoB, G32, Hx,
             gam, W1fBD, T64, W2fBD, WrA, WrB, VA1, VA2, VB1, VB2, ones80):
    nbs = lambda idx: (idx, 0)
    return pl.pallas_call(
        _n2_body,
        grid=(N // NB,),
        in_specs=[
            pl.BlockSpec((NB, HROW), nbs),
            pl.BlockSpec((NB, HROW), lambda idx: (idx + N // NB, 0)),
            pl.BlockSpec((NB, L * C), nbs), pl.BlockSpec((NB, L * C), nbs),
            _full((HROW, HROW)), _full((HROW, HROW)),
            _full((HROW, L * C)), _full((HROW, L * C)),
            _full((L * C, L)), _full((L, L * C)), _full((1, L * C)),
            _full((L * C, L * 2 * C)), _full((2 * C, L * 2 * C)),
            _full((L * 2 * C, L * C)),
            _full((L * C, L * C)), _full((L * C, L * C)),
            _full((L * C, HROW)), _full((L * C, HROW)),
            _full((L * C, HROW)), _full((L * C, HROW)), _full((1, HROW)),
        ],
        out_specs=[
            pl.BlockSpec((NB, L * C), nbs), pl.BlockSpec((NB, L * C), nbs),
            pl.BlockSpec((NB, C), nbs), pl.BlockSpec((NB, C), nbs),
            pl.BlockSpec((2, NB, HROW), lambda idx: (0, idx, 0)),
        ],
        out_shape=[
            jax.ShapeDtypeStruct((N, L * C), f32),
            jax.ShapeDtypeStruct((N, L * C), f32),
            jax.ShapeDtypeStruct((N, C), f32),
            jax.ShapeDtypeStruct((N, C), f32),
            jax.ShapeDtypeStruct((2, N, HROW), f32),
        ],
    )(out2, out2, node2d, res2d, SELA, SELB, WoA, WoB, G32, Hx, gam,
      W1fBD, T64, W2fBD, WrA, WrB, VA1, VA2, VB1, VB2, ones80)


def kernel(res_emb, t, edge_s, kappa, W1, b1, W2, b2, W_emb, b_emb, Wq, Wk,
           We_attn, Wv, Wo, gamma, W1f, W2f, Wres, We1, seg_ids, edge_index):
    res2d = res_emb.reshape(N, L * C).astype(f32)
    src = edge_index[0].astype(i32).reshape(NCH, CHUNK)
    dst = edge_index[1].astype(i32).reshape(NCH, CHUNK)
    srcdst = jnp.concatenate([src, dst], axis=0)
    seg2d = seg_ids.reshape(N, 1).astype(i32)
    t_pad = jnp.pad(t.astype(f32), (0, 6)).reshape(56, 1)
    kap = kappa.reshape(1, HT // 2).astype(f32)

    eye9 = jnp.eye(L, dtype=f32)

    def bd9(A):
        return jnp.kron(eye9, A.astype(f32))

    _r2 = np.kron(np.eye(H), np.ones((1, DV)))
    S16s = jnp.asarray(
        np.kron(np.eye(H), np.ones((DA, 1))) @ _r2 / np.sqrt(DA), f32)
    G32 = jnp.asarray(np.kron(np.eye(L), np.ones((C, 1))) / C, f32)
    Hx = jnp.asarray(np.kron(np.eye(L), np.ones((1, C))), f32)
    T64 = jnp.asarray(np.kron(np.ones((1, L)), np.eye(2 * C)), f32)
    selA = np.zeros((HROW, HROW), np.float32)
    selB = np.zeros((HROW, HROW), np.float32)
    for tt in range(HROW):
        selA[64 + tt % 16, tt] = 1.0
    for tt in range(64):
        selB[64 + tt % 16, tt] = 1.0
    for tt in range(64, HROW):
        selB[tt, tt] = 1.0
    SELA = jnp.asarray(selA, f32)
    SELB = jnp.asarray(selB, f32)
    ones80 = jnp.concatenate(
        [jnp.zeros((1, 64), f32), jnp.ones((1, 16), f32)], axis=1)
    zeros80 = jnp.zeros((N, HROW), f32)
    zpad = jnp.zeros((C, DE), f32)

    def vsplit(i):
        A = bd9(Wv[i][:C])
        Bm = bd9(Wv[i][C:])
        z = jnp.zeros((L * C, 16), f32)
        return (A[:, :HROW], jnp.concatenate([A[:, HROW:], z], 1),
                Bm[:, :HROW], jnp.concatenate([Bm[:, HROW:], z], 1))

    def wo_split(i):
        W = bd9(Wo[i])
        return W[:HROW], jnp.concatenate(
            [W[HROW:], jnp.zeros((16, L * C), f32)], 0)

    VA1, VA2, VB1, VB2 = vsplit(0)
    l0, v2 = _embed_call(t_pad, kap, W1.astype(f32), b1.reshape(1, 128),
                         W2.astype(f32), b2.reshape(1, HT),
                         W_emb[C:].astype(f32), b_emb.reshape(1, C), seg2d,
                         res2d, VA1, VA2, VB1[:C], VB2[:C], ones80)
    node2d = jnp.concatenate([l0, jnp.zeros((N, (L - 1) * C), f32)], axis=1)
    s = jnp.concatenate([res2d[:, :C], l0], axis=1)
    ef = edge_s.astype(f32)

    R2x = jnp.asarray(_r2, f32)
    gsout = _gs_call(s, srcdst)
    exx = _e1_call(gsout, ef, Wq[0], Wk[0], We_attn[0] @ R2x, S16s)

    for i in range(NL):
        vab = v2.reshape(2 * N, HROW)
        out2 = _agg_call(vab, exx, src, dst, zeros80)
        WoA, WoB = wo_split(i)
        VA1, VA2, VB1, VB2 = vsplit(min(i + 1, NL - 1))
        res2d, node2d, sa, sb, v2 = _n2_call(
            out2, node2d, res2d, SELA, SELB, WoA, WoB, G32, Hx,
            jnp.tile(gamma[i].reshape(1, C), (1, L)),
            bd9(W1f[i]), T64, bd9(W2f[i]), bd9(Wres[i][:C]),
            bd9(Wres[i][C:]), VA1, VA2, VB1, VB2, ones80)
        s = jnp.concatenate([sa, sb], axis=1)
        if i < NL - 1:
            gsout = _gs_call(s, srcdst)
            ef, exx = _ef_call(
                ef, gsout, We1[i][:DE],
                jnp.concatenate([zpad, We1[i][DE:DE + C]], axis=0),
                jnp.concatenate([zpad, We1[i][DE + C:]], axis=0),
                Wq[i + 1], Wk[i + 1], We_attn[i + 1] @ R2x, S16s)

    return res2d.reshape(N, L, C)
